# baseline (speedup 1.0000x reference)
import jax
import jax.numpy as jnp
from jax.experimental import pallas as pl
from jax.experimental.pallas import tpu as pltpu

_NUM_BINS = 256
_MIN_VAL = 0.0
_MAX_VAL = 1.0
_LANES = 128
_BINS_PER_PASS = 16
_ROWS_PER_STEP = 8


def _hist_kernel(x_ref, o_ref):
    inv_bw = (_NUM_BINS - 1) / (_MAX_VAL - _MIN_VAL)
    rows = x_ref.shape[1]
    n_pixels = rows * _LANES
    n_slabs = rows // _ROWS_PER_STEP
    shape3 = (_BINS_PER_PASS, _ROWS_PER_STEP, _LANES)

    parts = []
    for bin_base in range(0, _NUM_BINS, _BINS_PER_PASS):
        bins = (jax.lax.broadcasted_iota(jnp.int32, shape3, 0)
                .astype(jnp.float32) + float(bin_base))

        def body(i, acc, bins=bins):
            slab = x_ref[0, pl.ds(i * _ROWS_PER_STEP, _ROWS_PER_STEP), :]
            t = (slab - _MIN_VAL) * inv_bw
            x3 = jnp.broadcast_to(t[None, :, :], shape3)
            return acc + jnp.minimum(jnp.abs(x3 - bins), 1.0)

        acc0 = jnp.zeros(shape3, jnp.float32)
        acc = jax.lax.fori_loop(0, n_slabs, body, acc0, unroll=2)
        parts.append(jnp.sum(jnp.sum(acc, axis=1), axis=1))

    total = jnp.concatenate(parts)
    o_ref[...] = (float(n_pixels) - total).reshape(1, 1, _NUM_BINS)


def kernel(images_batch, bin_centers):
    del bin_centers
    b = images_batch.shape[0]
    n = images_batch.shape[1] * images_batch.shape[2] * images_batch.shape[3]
    rows = n // _LANES
    x = images_batch.reshape(b, rows, _LANES)
    out = pl.pallas_call(
        _hist_kernel,
        out_shape=jax.ShapeDtypeStruct((b, 1, _NUM_BINS), jnp.float32),
        grid=(b,),
        in_specs=[pl.BlockSpec((1, rows, _LANES), lambda i: (i, 0, 0))],
        out_specs=pl.BlockSpec((1, 1, _NUM_BINS), lambda i: (i, 0, 0)),
        compiler_params=pltpu.CompilerParams(
            dimension_semantics=("parallel",),
        ),
    )(x)
    return out.reshape(b, _NUM_BINS)
